# Initial kernel scaffold; baseline (speedup 1.0000x reference)
import jax
import jax.numpy as jnp
from jax.experimental import pallas as pl


def kernel(idx, targets, table):
    raise NotImplementedError("write your pallas kernel here")


# speedup vs baseline: 2.1525x; 2.1525x over previous
import functools

import jax
import jax.numpy as jnp
from jax import lax
from jax.experimental import pallas as pl
from jax.experimental.pallas import tpu as pltpu
from jax.experimental.pallas import tpu_sc as plsc

V = 8192
N = 16384
NC, NS = 2, 16
NW = NC * NS
CHUNK = N // NW
G = 4
NG = CHUNK // G
NP = NG // 2


def _gather_body(idx_hbm, table_hbm, out_hbm, idx_v, buf_a, buf_b,
                 gs_a, gs_b, ws_a, ws_b):
    wid = lax.axis_index("s") * NC + lax.axis_index("c")
    base = wid * CHUNK
    pltpu.sync_copy(idx_hbm.at[wid], idx_v)

    def gather(g, buf, sem):
        return pltpu.make_async_copy(
            table_hbm.at[idx_v.at[g]], buf, sem)

    def write(g, buf, sem):
        return pltpu.make_async_copy(
            buf, out_hbm.at[pl.ds(base + g * G, G)], sem)

    gather(0, buf_a, gs_a).start()
    gather(1, buf_b, gs_b).start()

    def body(p, carry):
        g0 = 2 * p
        gather(g0, buf_a, gs_a).wait()
        write(g0, buf_a, ws_a).start()
        gather(g0 + 1, buf_b, gs_b).wait()
        write(g0 + 1, buf_b, ws_b).start()

        @pl.when(p + 1 < NP)
        def _():
            write(g0, buf_a, ws_a).wait()
            gather(g0 + 2, buf_a, gs_a).start()
            write(g0 + 1, buf_b, ws_b).wait()
            gather(g0 + 3, buf_b, gs_b).start()

        return carry

    lax.fori_loop(0, NP, body, 0)
    write(NG - 2, buf_a, ws_a).wait()
    write(NG - 1, buf_b, ws_b).wait()


_sc_gather = functools.partial(
    pl.kernel,
    out_type=jax.ShapeDtypeStruct((N, V), jnp.float32),
    mesh=plsc.VectorSubcoreMesh(core_axis_name="c", subcore_axis_name="s"),
    scratch_types=[
        pltpu.VMEM((NG, G), jnp.int32),
        pltpu.VMEM((G, V), jnp.float32),
        pltpu.VMEM((G, V), jnp.float32),
        pltpu.SemaphoreType.DMA,
        pltpu.SemaphoreType.DMA,
        pltpu.SemaphoreType.DMA,
        pltpu.SemaphoreType.DMA,
    ],
)(_gather_body)


BR = 256
NBLK = N // BR


def _loss_body(tgt_ref, x_ref, out_ref, acc_ref):
    i = pl.program_id(0)
    x = x_ref[...]
    t = tgt_ref[0, 0, :]
    m = jnp.max(x, axis=1, keepdims=True)
    s = jnp.sum(jnp.exp(x - m), axis=1)
    lse = m[:, 0] + jnp.log(s)
    col = lax.broadcasted_iota(jnp.int32, (BR, V), 1)
    picked = jnp.sum(jnp.where(col == t[:, None], x, 0.0), axis=1)
    blocksum = jnp.sum(lse - picked)

    @pl.when(i == 0)
    def _():
        acc_ref[0] = 0.0

    acc_ref[0] += blocksum

    @pl.when(i == NBLK - 1)
    def _():
        out_ref[...] = jnp.reshape(acc_ref[0] * (1.0 / N), (1, 1))


_tc_loss = pl.pallas_call(
    _loss_body,
    grid=(NBLK,),
    in_specs=[
        pl.BlockSpec((1, 1, BR), lambda i: (i, 0, 0)),
        pl.BlockSpec((BR, V), lambda i: (i, 0)),
    ],
    out_specs=pl.BlockSpec((1, 1), lambda i: (0, 0)),
    out_shape=jax.ShapeDtypeStruct((1, 1), jnp.float32),
    scratch_shapes=[pltpu.SMEM((1,), jnp.float32)],
)


def kernel(idx, targets, table):
    idx_grp = idx.reshape(NW, NG, G)
    logits2d = _sc_gather(idx_grp, table)
    tgt3d = targets.reshape(NBLK, 1, BR)
    loss = _tc_loss(tgt3d, logits2d)[0, 0]
    return (logits2d.reshape(idx.shape[0], idx.shape[1], V), loss)
